# Initial kernel scaffold; baseline (speedup 1.0000x reference)
import jax
import jax.numpy as jnp
from jax.experimental import pallas as pl


def kernel(edge_index, node_attr, edge_attr, coord_diff, coord_cross, coord_vertical, params):
    raise NotImplementedError("write your pallas kernel here")


# speedup vs baseline: 10.2045x; 10.2045x over previous
import functools

import jax
import jax.numpy as jnp
import numpy as np
from jax import lax
from jax.experimental import pallas as pl
from jax.experimental.pallas import tpu as pltpu
from jax.experimental.pallas import tpu_sc as plsc

N_NODES = 10000
NPAD = 10240
N_EDGES = 320000
HID = 128
HEADS = 8
HDIM = 16

_NC = 2
_NS = 16
_NW = _NC * _NS
_CW = 80
_NROWS = N_EDGES // _CW
_RPW = _NROWS // _NW

_BM = 512
_HIGH = jax.lax.Precision.HIGHEST


def _np_consts():
    rsum = np.zeros((HID, 16), np.float32)
    for d in range(HID):
        rsum[d, d // HDIM] = 1.0
    rbc = np.zeros((16, HID), np.float32)
    for d in range(HID):
        rbc[d // HDIM, d] = 1.0
    ball = np.zeros((16, 48), np.float32)
    pall = np.zeros((16, 48), np.float32)
    s48 = np.zeros((48, 16), np.float32)
    for i in range(3):
        for j in range(3):
            ball[i, 16 * i + j] = 1.0
            pall[3 * i + j, 16 * i + j] = 1.0
            s48[16 * i + j, j] = 1.0
    e3 = np.zeros((16, 16), np.float32)
    e3[3, :] = 1.0
    return rsum, rbc, ball, pall, s48, e3


def _tc_linear(x, w, b, adds=(), act=None):
    m, k = x.shape
    o = w.shape[1]
    n_adds = len(adds)

    def body(x_ref, w_ref, b_ref, *rest):
        o_ref = rest[-1]
        y = jnp.dot(x_ref[...], w_ref[...], preferred_element_type=jnp.float32,
                    precision=_HIGH) + b_ref[...]
        for a in rest[:n_adds]:
            y = y + a[...]
        if act is not None:
            y = act(y)
        o_ref[...] = y

    in_specs = [pl.BlockSpec((_BM, k), lambda i: (i, 0)),
                pl.BlockSpec((k, o), lambda i: (0, 0)),
                pl.BlockSpec((1, o), lambda i: (0, 0))]
    in_specs += [pl.BlockSpec((_BM, o), lambda i: (i, 0)) for _ in adds]
    return pl.pallas_call(
        body, grid=(m // _BM,), in_specs=in_specs,
        out_specs=pl.BlockSpec((_BM, o), lambda i: (i, 0)),
        out_shape=jax.ShapeDtypeStruct((m, o), jnp.float32))(x, w, b, *adds)


def _tc_ln_res(x, res, g, b, act=None):
    m, k = x.shape

    def body(x_ref, r_ref, g_ref, b_ref, o_ref):
        xv = x_ref[...]
        mu = jnp.mean(xv, axis=1, keepdims=True)
        var = jnp.mean((xv - mu) ** 2, axis=1, keepdims=True)
        y = (xv - mu) / jnp.sqrt(var + 1e-5) * g_ref[...] + b_ref[...]
        y = r_ref[...] + y
        if act is not None:
            y = act(y)
        o_ref[...] = y

    return pl.pallas_call(
        body, grid=(m // _BM,),
        in_specs=[pl.BlockSpec((_BM, k), lambda i: (i, 0)),
                  pl.BlockSpec((_BM, k), lambda i: (i, 0)),
                  pl.BlockSpec((1, k), lambda i: (0, 0)),
                  pl.BlockSpec((1, k), lambda i: (0, 0))],
        out_specs=pl.BlockSpec((_BM, k), lambda i: (i, 0)),
        out_shape=jax.ShapeDtypeStruct((m, k), jnp.float32))(x, res, g, b)


def _tc_alpha(qd, ks, e, rsum):
    m = qd.shape[0]

    def body(q_ref, k_ref, e_ref, r_ref, a_ref, gm_ref):
        i = pl.program_id(0)
        p = q_ref[...] * (k_ref[...] + e_ref[...])
        a = jnp.dot(p, r_ref[...], preferred_element_type=jnp.float32,
                    precision=_HIGH) * 0.25
        a_ref[...] = a

        @pl.when(i == 0)
        def _():
            gm_ref[...] = jnp.full((1, 16), -1e30, jnp.float32)

        gm_ref[...] = jnp.maximum(gm_ref[...], jnp.max(a, axis=0, keepdims=True))

    return pl.pallas_call(
        body, grid=(m // _BM,),
        in_specs=[pl.BlockSpec((_BM, HID), lambda i: (i, 0)),
                  pl.BlockSpec((_BM, HID), lambda i: (i, 0)),
                  pl.BlockSpec((_BM, HID), lambda i: (i, 0)),
                  pl.BlockSpec((HID, 16), lambda i: (0, 0))],
        out_specs=[pl.BlockSpec((_BM, 16), lambda i: (i, 0)),
                   pl.BlockSpec((1, 16), lambda i: (0, 0))],
        out_shape=[jax.ShapeDtypeStruct((m, 16), jnp.float32),
                   jax.ShapeDtypeStruct((1, 16), jnp.float32)])(qd, ks, e, rsum)


def _tc_ex(alpha, gmax):
    m = alpha.shape[0]

    def body(a_ref, g_ref, o_ref):
        o_ref[...] = jnp.exp(a_ref[...] - g_ref[...])

    return pl.pallas_call(
        body, grid=(m // _BM,),
        in_specs=[pl.BlockSpec((_BM, 16), lambda i: (i, 0)),
                  pl.BlockSpec((1, 16), lambda i: (0, 0))],
        out_specs=pl.BlockSpec((_BM, 16), lambda i: (i, 0)),
        out_shape=jax.ShapeDtypeStruct((m, 16), jnp.float32))(alpha, gmax)


def _tc_bcast2(a, b, rbc):
    m = a.shape[0]

    def body(a_ref, b_ref, r_ref, o_ref):
        o_ref[...] = jnp.dot(a_ref[...] + b_ref[...], r_ref[...],
                             preferred_element_type=jnp.float32, precision=_HIGH)

    return pl.pallas_call(
        body, grid=(m // _BM,),
        in_specs=[pl.BlockSpec((_BM, 16), lambda i: (i, 0)),
                  pl.BlockSpec((_BM, 16), lambda i: (i, 0)),
                  pl.BlockSpec((16, HID), lambda i: (0, 0))],
        out_specs=pl.BlockSpec((_BM, HID), lambda i: (i, 0)),
        out_shape=jax.ShapeDtypeStruct((m, HID), jnp.float32))(a, b, rbc)


def _tc_msg(vs, e, ex, dend, rbc):
    m = vs.shape[0]

    def body(v_ref, e_ref, x_ref, d_ref, r_ref, o_ref):
        num = jnp.dot(x_ref[...], r_ref[...], preferred_element_type=jnp.float32,
                      precision=_HIGH)
        o_ref[...] = (v_ref[...] + e_ref[...]) * (num / d_ref[...])

    return pl.pallas_call(
        body, grid=(m // _BM,),
        in_specs=[pl.BlockSpec((_BM, HID), lambda i: (i, 0)),
                  pl.BlockSpec((_BM, HID), lambda i: (i, 0)),
                  pl.BlockSpec((_BM, 16), lambda i: (i, 0)),
                  pl.BlockSpec((_BM, HID), lambda i: (i, 0)),
                  pl.BlockSpec((16, HID), lambda i: (0, 0))],
        out_specs=pl.BlockSpec((_BM, HID), lambda i: (i, 0)),
        out_shape=jax.ShapeDtypeStruct((m, HID), jnp.float32))(vs, e, ex, dend, rbc)


def _tc_basis_h(a, b, c, w1n):
    m = a.shape[0]

    def body(a_ref, b_ref, c_ref, w_ref, o_ref):
        y = jnp.dot(a_ref[...] + b_ref[...], w_ref[...],
                    preferred_element_type=jnp.float32, precision=_HIGH)
        o_ref[...] = jax.nn.silu(y + c_ref[...])

    return pl.pallas_call(
        body, grid=(m // _BM,),
        in_specs=[pl.BlockSpec((_BM, HID), lambda i: (i, 0)),
                  pl.BlockSpec((_BM, HID), lambda i: (i, 0)),
                  pl.BlockSpec((_BM, 64), lambda i: (i, 0)),
                  pl.BlockSpec((HID, 64), lambda i: (0, 0))],
        out_specs=pl.BlockSpec((_BM, 64), lambda i: (i, 0)),
        out_shape=jax.ShapeDtypeStruct((m, 64), jnp.float32))(a, b, c, w1n)


def _tc_mix(h, c9, w2p, b2p, ball, pall, s48):
    m = h.shape[0]

    def body(h_ref, c_ref, w_ref, b_ref, bl_ref, pl_ref, s_ref, o_ref):
        dc = jnp.dot(h_ref[...], w_ref[...], preferred_element_type=jnp.float32,
                     precision=_HIGH) + b_ref[...]
        dcb = jnp.dot(dc, bl_ref[...], preferred_element_type=jnp.float32,
                      precision=_HIGH)
        cp = jnp.dot(c_ref[...], pl_ref[...], preferred_element_type=jnp.float32,
                     precision=_HIGH)
        mix = jnp.dot(dcb * cp, s_ref[...], preferred_element_type=jnp.float32,
                      precision=_HIGH)
        lane = jax.lax.broadcasted_iota(jnp.int32, (_BM, 16), 1)
        o_ref[...] = mix + jnp.where(lane == 3, 1.0, 0.0)

    return pl.pallas_call(
        body, grid=(m // _BM,),
        in_specs=[pl.BlockSpec((_BM, 64), lambda i: (i, 0)),
                  pl.BlockSpec((_BM, 16), lambda i: (i, 0)),
                  pl.BlockSpec((64, 16), lambda i: (0, 0)),
                  pl.BlockSpec((1, 16), lambda i: (0, 0)),
                  pl.BlockSpec((16, 48), lambda i: (0, 0)),
                  pl.BlockSpec((16, 48), lambda i: (0, 0)),
                  pl.BlockSpec((48, 16), lambda i: (0, 0))],
        out_specs=pl.BlockSpec((_BM, 16), lambda i: (i, 0)),
        out_shape=jax.ShapeDtypeStruct((m, 16), jnp.float32))(
            h, c9, w2p, b2p, ball, pall, s48)


def _tc_grad(s0a, s0b, s1a, s1b, e3):
    m = s0a.shape[0]

    def body(a_ref, b_ref, c_ref, d_ref, e_ref, o_ref):
        s0 = a_ref[...] + b_ref[...]
        s1 = c_ref[...] + d_ref[...]
        cnt = jnp.maximum(jnp.dot(s0, e_ref[...], preferred_element_type=jnp.float32,
                                  precision=_HIGH), 1.0)
        o_ref[...] = s0 / cnt + s1 / cnt

    return pl.pallas_call(
        body, grid=(m // _BM,),
        in_specs=[pl.BlockSpec((_BM, 16), lambda i: (i, 0))] * 4 +
                 [pl.BlockSpec((16, 16), lambda i: (0, 0))],
        out_specs=pl.BlockSpec((_BM, 16), lambda i: (i, 0)),
        out_shape=jax.ShapeDtypeStruct((m, 16), jnp.float32))(s0a, s0b, s1a, s1b, e3)


def _sc_gather(table, idxr):
    t_rows, d = table.shape
    nrows, cw = idxr.shape
    mesh = plsc.VectorSubcoreMesh(core_axis_name="c", subcore_axis_name="s")

    @functools.partial(
        pl.kernel, mesh=mesh,
        out_type=jax.ShapeDtypeStruct((nrows * cw, d), jnp.float32),
        scratch_types=[pltpu.VMEM((cw,), jnp.int32),
                       pltpu.VMEM((cw, d), jnp.float32),
                       pltpu.SemaphoreType.DMA])
    def k(table_h, idx_h, out_h, idx_v, rows_v, sem):
        wid = lax.axis_index("s") * _NC + lax.axis_index("c")
        rbase = wid * _RPW

        def step(j, carry):
            r = rbase + j
            pltpu.sync_copy(idx_h.at[r], idx_v)
            pltpu.async_copy(table_h.at[idx_v], rows_v, sem).wait()
            pltpu.sync_copy(rows_v, out_h.at[pl.ds(r * cw, cw)])
            return carry

        lax.fori_loop(0, _RPW, step, 0)

    return k(table, idxr)


def _sc_scatter_add(vals, idxr, zeros):
    e, d = vals.shape
    nrows, cw = idxr.shape
    npad = zeros.shape[0]
    rps = npad // _NS
    mesh = plsc.VectorSubcoreMesh(core_axis_name="c", subcore_axis_name="s")

    @functools.partial(
        pl.kernel, mesh=mesh,
        out_type=jax.ShapeDtypeStruct((_NC, npad, d), jnp.float32),
        scratch_types=[pltpu.VMEM((cw,), jnp.int32),
                       pltpu.VMEM((cw, d), jnp.float32),
                       pltpu.VMEM_SHARED((npad, d), jnp.float32)])
    def k(vals_h, idx_h, zeros_h, out_h, idx_v, val_v, table):
        cid = lax.axis_index("c")
        sid = lax.axis_index("s")
        wid = sid * _NC + cid
        rbase = wid * _RPW

        pltpu.sync_copy(zeros_h.at[pl.ds(sid * rps, rps)],
                        table.at[pl.ds(sid * rps, rps)])
        plsc.subcore_barrier()

        def step(j, carry):
            r = rbase + j
            pltpu.sync_copy(idx_h.at[r], idx_v)
            pltpu.sync_copy(vals_h.at[pl.ds(r * cw, cw)], val_v)
            pltpu.sync_copy(val_v, table.at[idx_v], add=True)
            return carry

        lax.fori_loop(0, _RPW, step, 0)
        plsc.subcore_barrier()
        pltpu.sync_copy(table.at[pl.ds(sid * rps, rps)],
                        out_h.at[cid, pl.ds(sid * rps, rps)])

    return k(vals, idxr, zeros)


def _gat_layer(x, e, srcr, dstr, p, consts, act):
    rsum, rbc, zeros16, zeros128 = consts
    q = _tc_linear(x, p['Wq'].T, p['bq'].reshape(1, -1))
    k = _tc_linear(x, p['Wk'].T, p['bk'].reshape(1, -1))
    v = _tc_linear(x, p['Wv'].T, p['bv'].reshape(1, -1))

    qd = _sc_gather(q, dstr)
    ks = _sc_gather(k, srcr)
    vs = _sc_gather(v, srcr)

    alpha, gmax = _tc_alpha(qd, ks, e, rsum)
    ex = _tc_ex(alpha, gmax)
    denp = _sc_scatter_add(ex, dstr, zeros16)
    den128 = _tc_bcast2(denp[0], denp[1], rbc)
    dend = _sc_gather(den128, dstr)
    msg = _tc_msg(vs, e, ex, dend, rbc)
    msgp = _sc_scatter_add(msg, dstr, zeros128)

    o = _tc_linear(x, p['Ws'].T, p['bs'].reshape(1, -1), adds=(msgp[0], msgp[1]))
    x1 = _tc_ln_res(o, x, p['n1g'].reshape(1, -1), p['n1b'].reshape(1, -1))
    h1 = _tc_linear(x1, p['f1'].T, p['fb1'].reshape(1, -1), act=jax.nn.silu)
    y = _tc_linear(h1, p['f2'].T, p['fb2'].reshape(1, -1))
    return _tc_ln_res(y, x1, p['n2g'].reshape(1, -1), p['n2b'].reshape(1, -1),
                      act=act)


def kernel(edge_index, node_attr, edge_attr, coord_diff, coord_cross,
           coord_vertical, params):
    src = edge_index[0].astype(jnp.int32)
    dst = edge_index[1].astype(jnp.int32)
    srcr = src.reshape(_NROWS, _CW)
    dstr = dst.reshape(_NROWS, _CW)

    rsum_np, rbc_np, ball_np, pall_np, s48_np, e3_np = _np_consts()
    rsum = jnp.asarray(rsum_np)
    rbc = jnp.asarray(rbc_np)
    ball = jnp.asarray(ball_np)
    pall = jnp.asarray(pall_np)
    s48 = jnp.asarray(s48_np)
    e3 = jnp.asarray(e3_np)
    zeros16 = jnp.zeros((NPAD, 16), jnp.float32)
    zeros128 = jnp.zeros((NPAD, HID), jnp.float32)
    consts = (rsum, rbc, zeros16, zeros128)

    x = jnp.pad(node_attr, ((0, NPAD - N_NODES), (0, 0)))
    c9 = jnp.pad(jnp.concatenate([coord_diff, coord_cross, coord_vertical],
                                 axis=1), ((0, 0), (0, 7)))

    mix_partials = []
    for m in range(2):
        for c in range(2):
            p = params['gat_%d_%d' % (m, c)]
            e = _tc_linear(edge_attr, p['We'].T,
                           jnp.zeros((1, HID), jnp.float32))
            act = jax.nn.silu if c < 1 else None
            x = _gat_layer(x, e, srcr, dstr, p, consts, act)
        bp = params['basis_%d' % m]
        w1n = bp['w1'][:, :HID].T
        w1e = bp['w1'][:, HID:].T
        w2p = jnp.pad(bp['w2'].T, ((0, 0), (0, 13)))
        b2p = jnp.pad(bp['b2'], (0, 13)).reshape(1, 16)
        eeb = _tc_linear(edge_attr, w1e, bp['b1'].reshape(1, -1))
        nfs = _sc_gather(x, srcr)
        nfd = _sc_gather(x, dstr)
        h = _tc_basis_h(nfs, nfd, eeb, w1n)
        mix = _tc_mix(h, c9, w2p, b2p, ball, pall, s48)
        mp = _sc_scatter_add(mix, dstr, zeros16)
        mix_partials.append(mp)

    grad16 = _tc_grad(mix_partials[0][0], mix_partials[0][1],
                      mix_partials[1][0], mix_partials[1][1], e3)
    return x[:N_NODES], grad16[:N_NODES, :3]
